# baseline (speedup 1.0000x reference)
import functools

import jax
import jax.numpy as jnp
from jax import lax
from jax.experimental import pallas as pl
from jax.experimental.pallas import tpu as pltpu
from jax.experimental.pallas import tpu_sc as plsc

_COPIES = (
    (0, 0, 0, 256),
    (0, 512, 256, 256),
    (1, 128, 512, 256),
    (1, 768, 768, 256),
)


def kernel(tensor_a, tensor_b):
    rows, _ = tensor_a.shape
    out_cols = sum(w for _, _, _, w in _COPIES)

    info = plsc.get_sparse_core_info()
    num_workers = info.num_cores * info.num_subcores
    rows_per = rows // num_workers

    mesh = plsc.VectorSubcoreMesh(core_axis_name="c", subcore_axis_name="s")

    @functools.partial(
        pl.kernel,
        mesh=mesh,
        out_type=jax.ShapeDtypeStruct((rows, out_cols), jnp.float32),
    )
    def run(a_hbm, b_hbm, out_hbm):
        wid = lax.axis_index("s") * info.num_cores + lax.axis_index("c")
        base = wid * rows_per
        srcs = (a_hbm, b_hbm)
        for which, src_col, dst_col, width in _COPIES:
            pltpu.sync_copy(
                srcs[which].at[pl.ds(base, rows_per), pl.ds(src_col, width)],
                out_hbm.at[pl.ds(base, rows_per), pl.ds(dst_col, width)],
            )

    return run(tensor_a, tensor_b)


# speedup vs baseline: 28.4508x; 28.4508x over previous
import functools

import jax
import jax.numpy as jnp
from jax import lax
from jax.experimental import pallas as pl
from jax.experimental.pallas import tpu as pltpu
from jax.experimental.pallas import tpu_sc as plsc

_COPIES = (
    (0, 0, 0, 256),
    (0, 512, 256, 256),
    (1, 128, 512, 256),
    (1, 768, 768, 256),
)

_NBUF = 3
_CHUNK = 32


def kernel(tensor_a, tensor_b):
    rows, _ = tensor_a.shape
    out_cols = sum(w for _, _, _, w in _COPIES)

    info = plsc.get_sparse_core_info()
    num_workers = info.num_cores * info.num_subcores
    rows_per = rows // num_workers
    n_chunks = rows_per // _CHUNK

    mesh = plsc.VectorSubcoreMesh(core_axis_name="c", subcore_axis_name="s")

    @functools.partial(
        pl.kernel,
        mesh=mesh,
        out_type=jax.ShapeDtypeStruct((rows, out_cols), jnp.float32),
        scratch_types=[
            pltpu.VMEM((_NBUF, _CHUNK, out_cols), jnp.float32),
            pltpu.SemaphoreType.DMA((_NBUF,)),
            pltpu.SemaphoreType.DMA((_NBUF,)),
        ],
    )
    def run(a_hbm, b_hbm, out_hbm, buf, sem_r, sem_w):
        wid = lax.axis_index("s") * info.num_cores + lax.axis_index("c")
        base = wid * rows_per
        srcs = (a_hbm, b_hbm)

        read_handles = [None] * n_chunks
        write_handles = [None] * n_chunks

        def start_reads(g):
            slot = g % _NBUF
            r0 = base + g * _CHUNK
            handles = []
            for which, src_col, dst_col, width in _COPIES:
                handles.append(
                    pltpu.make_async_copy(
                        srcs[which].at[pl.ds(r0, _CHUNK), pl.ds(src_col, width)],
                        buf.at[slot, :, pl.ds(dst_col, width)],
                        sem_r.at[slot],
                    )
                )
            for h in handles:
                h.start()
            read_handles[g] = handles

        def start_write(g):
            slot = g % _NBUF
            r0 = base + g * _CHUNK
            for h in read_handles[g]:
                h.wait()
            h = pltpu.make_async_copy(
                buf.at[slot], out_hbm.at[pl.ds(r0, _CHUNK)], sem_w.at[slot]
            )
            h.start()
            write_handles[g] = h

        for g in range(n_chunks + 1):
            if g < n_chunks:
                if g >= _NBUF:
                    write_handles[g - _NBUF].wait()
                start_reads(g)
            if g >= 1:
                start_write(g - 1)
        for g in range(n_chunks - _NBUF, n_chunks):
            write_handles[g].wait()

    return run(tensor_a, tensor_b)
